# Initial kernel scaffold; baseline (speedup 1.0000x reference)
import jax
import jax.numpy as jnp
from jax.experimental import pallas as pl


def kernel(m_ji, e_rbf, a_sbf, kj_idx, ji_idx, W_mkj, b_mkj, W_e1, W_e2, W_a1, W_a2, W_down, W_up, W_mji, b_mji, W_post, b_post, W_res00, b_res00, W_res01, b_res01, W_res10, b_res10, W_res11, b_res11, W_res20, b_res20, W_res21, b_res21):
    raise NotImplementedError("write your pallas kernel here")


# speedup vs baseline: 3.0144x; 3.0144x over previous
import jax
import jax.numpy as jnp
from jax import lax
from jax.experimental import pallas as pl
from jax.experimental.pallas import tpu as pltpu
from jax.experimental.pallas import tpu_sc as plsc

E = 320000
A = 960000
D = 128
IDIM = 64

NC = 2
NS = 16
NW = NC * NS

BE1 = 2000


def _prep_body(m_ref, e6_ref, wmkj_ref, bmkj_ref, we1_ref, we2_ref,
               mkj_out, et_out):
    m = m_ref[...]
    z = jnp.dot(m, wmkj_ref[...], preferred_element_type=jnp.float32)
    z = z + bmkj_ref[...]
    mkj_out[...] = z * jax.nn.sigmoid(z)
    we = jnp.dot(we1_ref[...], we2_ref[...], preferred_element_type=jnp.float32)
    et_out[...] = jnp.dot(e6_ref[...], we, preferred_element_type=jnp.float32)


def _prep(m_ji, e_rbf, W_mkj, b_mkj2, W_e1, W_e2):
    n = E // BE1
    return pl.pallas_call(
        _prep_body,
        grid=(n,),
        in_specs=[
            pl.BlockSpec((BE1, D), lambda i: (i, 0)),
            pl.BlockSpec((BE1, 6), lambda i: (i, 0)),
            pl.BlockSpec((D, D), lambda i: (0, 0)),
            pl.BlockSpec((1, D), lambda i: (0, 0)),
            pl.BlockSpec((6, 8), lambda i: (0, 0)),
            pl.BlockSpec((8, D), lambda i: (0, 0)),
        ],
        out_specs=[
            pl.BlockSpec((BE1, D), lambda i: (i, 0)),
            pl.BlockSpec((BE1, D), lambda i: (i, 0)),
        ],
        out_shape=[
            jax.ShapeDtypeStruct((E, D), jnp.float32),
            jax.ShapeDtypeStruct((E, D), jnp.float32),
        ],
        compiler_params=pltpu.CompilerParams(
            dimension_semantics=("parallel",)),
    )(m_ji, e_rbf, W_mkj, b_mkj2, W_e1, W_e2)


GC = 600
GG = 120
GCH = (A // NW) // GC


def _gather_body(mkj_hbm, et_hbm, kj_hbm, ji_hbm, gkj_hbm, gji_hbm,
                 idx_v, rows_v, sem):
    wid = lax.axis_index("s") * NC + lax.axis_index("c")
    base = wid * (A // NW)

    def chunk(c, carry):
        off = base + c * GC
        for k in range(GC // GG):
            pltpu.sync_copy(kj_hbm.at[pl.ds(off + k * GG, GG)], idx_v.at[k])
        cps = [pltpu.async_copy(mkj_hbm.at[idx_v.at[k]],
                                rows_v.at[pl.ds(k * GG, GG)], sem)
               for k in range(GC // GG)]
        for cp in cps:
            cp.wait()
        pltpu.sync_copy(rows_v, gkj_hbm.at[pl.ds(off, GC)])

        for k in range(GC // GG):
            pltpu.sync_copy(ji_hbm.at[pl.ds(off + k * GG, GG)], idx_v.at[k])
        cps = [pltpu.async_copy(et_hbm.at[idx_v.at[k]],
                                rows_v.at[pl.ds(k * GG, GG)], sem)
               for k in range(GC // GG)]
        for cp in cps:
            cp.wait()
        pltpu.sync_copy(rows_v, gji_hbm.at[pl.ds(off, GC)])
        return carry

    lax.fori_loop(0, GCH, chunk, 0)


def _gather(mkj_tab, e_tab, kj, ji):
    return pl.kernel(
        _gather_body,
        out_type=(jax.ShapeDtypeStruct((A, D), jnp.float32),
                  jax.ShapeDtypeStruct((A, D), jnp.float32)),
        mesh=plsc.VectorSubcoreMesh(core_axis_name="c", subcore_axis_name="s",
                                    num_cores=NC, num_subcores=NS),
        scratch_types=[
            pltpu.VMEM((GC // GG, GG), jnp.int32),
            pltpu.VMEM((GC, D), jnp.float32),
            pltpu.SemaphoreType.DMA,
        ],
    )(mkj_tab, e_tab, kj, ji)


BA3 = 3000


def _mid_body(gkj_ref, gji_ref, a49_ref, wa1_ref, wa2_ref, wd_ref, out_ref):
    prod = gkj_ref[...] * gji_ref[...]
    em = jnp.dot(prod, wd_ref[...], preferred_element_type=jnp.float32)
    em = em * jax.nn.sigmoid(em)
    a8 = jnp.dot(a49_ref[...], wa1_ref[...], preferred_element_type=jnp.float32)
    a64 = jnp.dot(a8, wa2_ref[...], preferred_element_type=jnp.float32)
    out_ref[...] = em * a64


def _mid(gkj, gji, a_sbf, W_a1, W_a2, W_down):
    n = A // BA3
    return pl.pallas_call(
        _mid_body,
        grid=(n,),
        in_specs=[
            pl.BlockSpec((BA3, D), lambda i: (i, 0)),
            pl.BlockSpec((BA3, D), lambda i: (i, 0)),
            pl.BlockSpec((BA3, 49), lambda i: (i, 0)),
            pl.BlockSpec((49, 8), lambda i: (0, 0)),
            pl.BlockSpec((8, IDIM), lambda i: (0, 0)),
            pl.BlockSpec((D, IDIM), lambda i: (0, 0)),
        ],
        out_specs=pl.BlockSpec((BA3, IDIM), lambda i: (i, 0)),
        out_shape=jax.ShapeDtypeStruct((A, IDIM), jnp.float32),
        compiler_params=pltpu.CompilerParams(
            dimension_semantics=("parallel",)),
    )(gkj, gji, a_sbf, W_a1, W_a2, W_down)


R4 = 16000
ROWS4 = 16384
CA = 96
NCH4 = A // CA
CPS = NCH4 // NS
PASSES = 10
EHALF = E // NC
ZB = 64


def _scatter_body(aggr_hbm, ji_hbm, out_hbm, idx_v, adj_v, rows_v, zb_v,
                  ob_v, acc_sh, sem):
    core = lax.axis_index("c")
    sid = lax.axis_index("s")
    zero16 = jnp.zeros((16,), jnp.float32)
    iota16 = lax.broadcasted_iota(jnp.int32, (16,), 0)

    def zr(r, carry):
        for cc in range(IDIM // 16):
            zb_v[r, pl.ds(cc * 16, 16)] = zero16
        return carry

    lax.fori_loop(0, ZB, zr, 0)

    def do_pass(p, carry):
        base = pl.multiple_of(core * EHALF + p * R4, 8)
        for b in range(16):
            pltpu.sync_copy(zb_v, acc_sh.at[pl.ds(sid * 1024 + b * ZB, ZB)])
        plsc.subcore_barrier()

        def chunk(gi, c1):
            off = pl.multiple_of((sid * CPS + gi) * CA, 8)
            pltpu.sync_copy(ji_hbm.at[pl.ds(off, CA)], idx_v)
            cp = pltpu.async_copy(aggr_hbm.at[pl.ds(off, CA)], rows_v, sem)

            def grp(k, c2):
                v = idx_v[pl.ds(k * 16, 16)]
                local = v - base
                inb = (local >= 0) & (local < R4)
                dump = R4 + k * 16 + iota16
                adj_v[pl.ds(k * 16, 16)] = jnp.where(inb, local, dump)
                return c2

            lax.fori_loop(0, CA // 16, grp, 0)
            cp.wait()
            pltpu.sync_copy(rows_v, acc_sh.at[adj_v], add=True)
            return c1

        lax.fori_loop(0, CPS, chunk, 0)
        plsc.subcore_barrier()
        for b in range(5):
            l0 = sid * 1000 + b * 200
            pltpu.sync_copy(acc_sh.at[pl.ds(l0, 200)], ob_v)
            pltpu.sync_copy(ob_v, out_hbm.at[pl.ds(base + l0, 200)])
        plsc.subcore_barrier()
        return carry

    lax.fori_loop(0, PASSES, do_pass, 0)


def _scatter(aggr, ji):
    return pl.kernel(
        _scatter_body,
        out_type=jax.ShapeDtypeStruct((E, IDIM), jnp.float32),
        mesh=plsc.VectorSubcoreMesh(core_axis_name="c", subcore_axis_name="s",
                                    num_cores=NC, num_subcores=NS),
        scratch_types=[
            pltpu.VMEM((CA,), jnp.int32),
            pltpu.VMEM((CA,), jnp.int32),
            pltpu.VMEM((CA, IDIM), jnp.float32),
            pltpu.VMEM((ZB, IDIM), jnp.float32),
            pltpu.VMEM((200, IDIM), jnp.float32),
            pltpu.VMEM_SHARED((ROWS4, IDIM), jnp.float32),
            pltpu.SemaphoreType.DMA,
        ],
    )(aggr, ji)


BE5 = 2000


def _silu(x):
    return x * jax.nn.sigmoid(x)


def _final_body(s_ref, m_ref, wup_ref, wmji_ref, bmji_ref, wpost_ref,
                bpost_ref, w00_ref, b00_ref, w01_ref, b01_ref,
                w10_ref, b10_ref, w11_ref, b11_ref,
                w20_ref, b20_ref, w21_ref, b21_ref, out_ref):
    m = m_ref[...]
    d = _silu(jnp.dot(s_ref[...], wup_ref[...],
                      preferred_element_type=jnp.float32))
    dm = _silu(jnp.dot(m, wmji_ref[...],
                       preferred_element_type=jnp.float32) + bmji_ref[...])
    x = d + dm

    def res(x, w1, b1, w2, b2):
        r = _silu(jnp.dot(x, w1[...], preferred_element_type=jnp.float32)
                  + b1[...])
        r = _silu(jnp.dot(r, w2[...], preferred_element_type=jnp.float32)
                  + b2[...])
        return r + x

    x = res(x, w00_ref, b00_ref, w01_ref, b01_ref)
    x = _silu(jnp.dot(x, wpost_ref[...],
                      preferred_element_type=jnp.float32) + bpost_ref[...]) + m
    x = res(x, w10_ref, b10_ref, w11_ref, b11_ref)
    x = res(x, w20_ref, b20_ref, w21_ref, b21_ref)
    out_ref[...] = x


def _final(summed, m_ji, W_up, W_mji, b_mji2, W_post, b_post2, rws):
    n = E // BE5
    wspec = pl.BlockSpec((D, D), lambda i: (0, 0))
    bspec = pl.BlockSpec((1, D), lambda i: (0, 0))
    rspecs = []
    for k in range(6):
        rspecs.extend([wspec, bspec])
    return pl.pallas_call(
        _final_body,
        grid=(n,),
        in_specs=[
            pl.BlockSpec((BE5, IDIM), lambda i: (i, 0)),
            pl.BlockSpec((BE5, D), lambda i: (i, 0)),
            pl.BlockSpec((IDIM, D), lambda i: (0, 0)),
            wspec, bspec, wspec, bspec,
        ] + rspecs,
        out_specs=pl.BlockSpec((BE5, D), lambda i: (i, 0)),
        out_shape=jax.ShapeDtypeStruct((E, D), jnp.float32),
        compiler_params=pltpu.CompilerParams(
            dimension_semantics=("parallel",)),
    )(summed, m_ji, W_up, W_mji, b_mji2, W_post, b_post2, *rws)


def kernel(m_ji, e_rbf, a_sbf, kj_idx, ji_idx,
           W_mkj, b_mkj, W_e1, W_e2, W_a1, W_a2, W_down, W_up,
           W_mji, b_mji, W_post, b_post,
           W_res00, b_res00, W_res01, b_res01,
           W_res10, b_res10, W_res11, b_res11,
           W_res20, b_res20, W_res21, b_res21):
    kj = kj_idx.astype(jnp.int32)
    ji = ji_idx.astype(jnp.int32)

    mkj_tab, e_tab = _prep(m_ji, e_rbf, W_mkj, b_mkj.reshape(1, D),
                           W_e1, W_e2)
    gkj, gji = _gather(mkj_tab, e_tab, kj, ji)
    aggr = _mid(gkj, gji, a_sbf, W_a1, W_a2, W_down)
    summed = jax.ops.segment_sum(aggr, ji, num_segments=E)
    rws = []
    for w, b in ((W_res00, b_res00), (W_res01, b_res01),
                 (W_res10, b_res10), (W_res11, b_res11),
                 (W_res20, b_res20), (W_res21, b_res21)):
        rws.extend([w, b.reshape(1, D)])
    return _final(summed, m_ji, W_up, W_mji, b_mji.reshape(1, D),
                  W_post, b_post.reshape(1, D), rws)
